# baseline (speedup 1.0000x reference)
import jax
import jax.numpy as jnp
from jax import lax
from jax.experimental import pallas as pl
from jax.experimental.pallas import tpu as pltpu
from jax.experimental.pallas import tpu_sc as plsc

B, T, D = 32, 2047, 1024
NC, NS = 2, 16
NW = NC * NS
TT = 8
NG = B // 8
DSC = 896
SC_ROWS = 2040
NTC = SC_ROWS // TT
NUNITS = NTC * NG
KOUT = (NUNITS + 2 * NW - 1) // (2 * NW)


def _sc_body(cache_t_hbm, order_hbm, out_hbm, ord_v, slab0, slab1, gsems, osems):
    wid = lax.axis_index("s") * NC + lax.axis_index("c")
    slabs = (slab0, slab1)

    pltpu.sync_copy(order_hbm, ord_v)
    lanes = lax.iota(jnp.int32, 16)
    zero = jnp.zeros((16,), jnp.int32)
    lo = ord_v[pl.ds(0, 16)]
    hi = ord_v[pl.ds(16, 16)]
    srcs = [
        jnp.max(jnp.where(lanes == i, lo, zero)) if i < 16
        else jnp.max(jnp.where(lanes == i - 16, hi, zero))
        for i in range(B)
    ]
    src_g = [srcs[i] // 8 for i in range(B)]
    src_b = [srcs[i] % 8 for i in range(B)]

    def params(u):
        g = u % NG
        tc = u // NG
        t0 = pl.multiple_of(tc * TT, TT)
        b0 = pl.multiple_of(g * 8, 8)
        return g, t0, b0

    def gather_copy(u, slot, tt):
        _, t0, b0 = params(u)
        return pltpu.make_async_copy(
            cache_t_hbm.at[t0 + tt, pl.ds(b0, 8), pl.ds(0, DSC)],
            slabs[slot].at[:, tt],
            gsems[slot],
        )

    def store_copy(u, slot, i):
        _, t0, _ = params(u)
        return pltpu.make_async_copy(
            slabs[slot].at[src_b[i]],
            out_hbm.at[i, pl.ds(t0, TT), pl.ds(0, DSC)],
            osems[slot],
        )

    def hit(u):
        g, _, _ = params(u)
        h = src_g[0] == g
        for i in range(1, B):
            h = h | (src_g[i] == g)
        return h

    def drain(u, slot):
        g, _, _ = params(u)
        for i in range(B):
            @pl.when(src_g[i] == g)
            def _(i=i):
                store_copy(u, slot, i).wait()

    def body(j, carry):
        for b in range(2):
            u = wid + NW * (2 * j + b)
            up = u - 2 * NW
            @pl.when((up >= 0) & (up < NUNITS) & hit(up))
            def _(up=up, b=b):
                drain(up, b)
            @pl.when((u < NUNITS) & hit(u))
            def _(u=u, b=b):
                for tt in range(TT):
                    gather_copy(u, b, tt).start()
        for b in range(2):
            u = wid + NW * (2 * j + b)
            @pl.when((u < NUNITS) & hit(u))
            def _(u=u, b=b):
                for tt in range(TT):
                    gather_copy(u, b, tt).wait()
                for i in range(B):
                    @pl.when(src_g[i] == params(u)[0])
                    def _(i=i, u=u, b=b):
                        store_copy(u, b, i).start()
        return carry

    lax.fori_loop(0, KOUT, body, 0)

    for b in range(2):
        u = wid + NW * (2 * (KOUT - 1) + b)
        @pl.when((u >= 0) & (u < NUNITS) & hit(u))
        def _(u=u, b=b):
            drain(u, b)


def _sc_bulk(cache_t, order):
    mesh = plsc.VectorSubcoreMesh(
        core_axis_name="c", subcore_axis_name="s", num_cores=NC, num_subcores=NS
    )
    return pl.kernel(
        _sc_body,
        out_type=jax.ShapeDtypeStruct((B, T + 1, D), jnp.float32),
        mesh=mesh,
        compiler_params=pltpu.CompilerParams(needs_layout_passes=False),
        scratch_types=[
            pltpu.VMEM((B,), jnp.int32),
            pltpu.VMEM((8, TT, DSC), jnp.float32),
            pltpu.VMEM((8, TT, DSC), jnp.float32),
            [pltpu.SemaphoreType.DMA for _ in range(2)],
            [pltpu.SemaphoreType.DMA for _ in range(2)],
        ],
    )(cache_t, order)


def _tc_strip_body(order_ref, cache_t_ref, prev_ref, out_ref):
    del prev_ref
    i = pl.program_id(1)
    src = order_ref[i]
    blk = cache_t_ref[:, pl.ds(src, 1), :]
    out_ref[0] = blk[:, 0, :]


def _tc_strip(cache_t, order, prev):
    grid_spec = pltpu.PrefetchScalarGridSpec(
        num_scalar_prefetch=1,
        grid=(16, B),
        in_specs=[
            pl.BlockSpec((128, B, 128), lambda tb, i, ord_ref: (tb, 0, 7)),
            pl.BlockSpec(memory_space=pl.ANY),
        ],
        out_specs=pl.BlockSpec((1, 128, 128), lambda tb, i, ord_ref: (i, tb, 7)),
    )
    return pl.pallas_call(
        _tc_strip_body,
        grid_spec=grid_spec,
        out_shape=jax.ShapeDtypeStruct((B, T + 1, D), jnp.float32),
        input_output_aliases={2: 0},
    )(order, cache_t, prev)


def _tc_tail_body(order_ref, cache_t_ref, s_ref, prev_ref, out_ref):
    del prev_ref
    i = pl.program_id(0)
    src = order_ref[i]
    blk = cache_t_ref[:, pl.ds(src, 1), :]
    out_ref[0, :7] = blk[:7, 0, :]
    out_ref[0, 7:8] = s_ref[0]


def _tc_tail(cache_t, s, order, prev):
    grid_spec = pltpu.PrefetchScalarGridSpec(
        num_scalar_prefetch=1,
        grid=(B,),
        in_specs=[
            pl.BlockSpec((8, B, D), lambda i, ord_ref: (T // 8, 0, 0)),
            pl.BlockSpec((1, 1, D), lambda i, ord_ref: (ord_ref[i], 0, 0)),
            pl.BlockSpec(memory_space=pl.ANY),
        ],
        out_specs=pl.BlockSpec((1, 8, D), lambda i, ord_ref: (i, T // 8, 0)),
    )
    return pl.pallas_call(
        _tc_tail_body,
        grid_spec=grid_spec,
        out_shape=jax.ShapeDtypeStruct((B, T + 1, D), jnp.float32),
        input_output_aliases={3: 0},
    )(order, cache_t, s, prev)


@jax.jit
def kernel(cache, s, order):
    cache_t = jnp.transpose(cache, (1, 0, 2))
    out = _sc_bulk(cache_t, order)
    out = _tc_strip(cache_t, order, out)
    return _tc_tail(cache_t, s, order, out)


# speedup vs baseline: 1.6567x; 1.6567x over previous
import jax
import jax.numpy as jnp
from jax import lax
from jax.experimental import pallas as pl
from jax.experimental.pallas import tpu as pltpu
from jax.experimental.pallas import tpu_sc as plsc

B, T, D = 32, 2047, 1024
NC, NS = 2, 16
NW = NC * NS
TT = 8
NG = B // 8
SC_ROWS = 2040
NTC = SC_ROWS // TT
NUNITS = NTC * NG
KMAX = (NUNITS + NW - 1) // NW


def _sc_body(cache_t_hbm, order_hbm, out_hbm, ord_v, slab, gsem, osem):
    wid = lax.axis_index("s") * NC + lax.axis_index("c")

    pltpu.sync_copy(order_hbm, ord_v)
    lanes = lax.iota(jnp.int32, 16)
    zero = jnp.zeros((16,), jnp.int32)
    lo = ord_v[pl.ds(0, 16)]
    hi = ord_v[pl.ds(16, 16)]
    srcs = [
        jnp.max(jnp.where(lanes == i, lo, zero)) if i < 16
        else jnp.max(jnp.where(lanes == i - 16, hi, zero))
        for i in range(B)
    ]

    def unit(k, carry):
        u = wid + NW * k
        valid = u < NUNITS
        g = u % NG
        tc = u // NG
        t0 = pl.multiple_of(tc * TT, TT)
        b0 = pl.multiple_of(g * 8, 8)

        conds = [valid & (srcs[i] // 8 == g) for i in range(B)]
        any_hit = conds[0]
        for i in range(1, B):
            any_hit = any_hit | conds[i]

        @pl.when(any_hit)
        def _():
            for tt in range(TT):
                pltpu.make_async_copy(
                    cache_t_hbm.at[t0 + tt, pl.ds(b0, 8)],
                    slab.at[:, tt],
                    gsem,
                ).start()
            for tt in range(TT):
                pltpu.make_async_copy(
                    cache_t_hbm.at[t0 + tt, pl.ds(b0, 8)],
                    slab.at[:, tt],
                    gsem,
                ).wait()
            for i in range(B):
                @pl.when(conds[i])
                def _(i=i):
                    pltpu.make_async_copy(
                        slab.at[srcs[i] % 8],
                        out_hbm.at[i, pl.ds(t0, TT)],
                        osem,
                    ).start()
            for i in range(B):
                @pl.when(conds[i])
                def _(i=i):
                    pltpu.make_async_copy(
                        slab.at[srcs[i] % 8],
                        out_hbm.at[i, pl.ds(t0, TT)],
                        osem,
                    ).wait()

        return carry

    lax.fori_loop(0, KMAX, unit, 0)


def _sc_bulk(cache_t, order):
    mesh = plsc.VectorSubcoreMesh(
        core_axis_name="c", subcore_axis_name="s", num_cores=NC, num_subcores=NS
    )
    return pl.kernel(
        _sc_body,
        out_type=jax.ShapeDtypeStruct((B, T + 1, D), jnp.float32),
        mesh=mesh,
        compiler_params=pltpu.CompilerParams(needs_layout_passes=False),
        scratch_types=[
            pltpu.VMEM((B,), jnp.int32),
            pltpu.VMEM((8, TT, D), jnp.float32),
            pltpu.SemaphoreType.DMA,
            pltpu.SemaphoreType.DMA,
        ],
    )(cache_t, order)


def _tc_tail_body(order_ref, cache_t_ref, s_ref, prev_ref, out_ref):
    del prev_ref
    i = pl.program_id(0)
    src = order_ref[i]
    blk = cache_t_ref[:, pl.ds(src, 1), :]
    out_ref[0, :7] = blk[:7, 0, :]
    out_ref[0, 7:8] = s_ref[0]


def _tc_tail(cache_t, s, order, prev):
    grid_spec = pltpu.PrefetchScalarGridSpec(
        num_scalar_prefetch=1,
        grid=(B,),
        in_specs=[
            pl.BlockSpec((8, B, D), lambda i, ord_ref: (T // 8, 0, 0)),
            pl.BlockSpec((1, 1, D), lambda i, ord_ref: (ord_ref[i], 0, 0)),
            pl.BlockSpec(memory_space=pl.ANY),
        ],
        out_specs=pl.BlockSpec((1, 8, D), lambda i, ord_ref: (i, T // 8, 0)),
    )
    return pl.pallas_call(
        _tc_tail_body,
        grid_spec=grid_spec,
        out_shape=jax.ShapeDtypeStruct((B, T + 1, D), jnp.float32),
        input_output_aliases={3: 0},
    )(order, cache_t, s, prev)


@jax.jit
def kernel(cache, s, order):
    cache_t = jnp.transpose(cache, (1, 0, 2))
    return _tc_tail(cache_t, s, order, _sc_bulk(cache_t, order))
